# baseline (speedup 1.0000x reference)
import jax
import jax.numpy as jnp
from jax.experimental import pallas as pl
from jax.experimental.pallas import tpu as pltpu

_K = 200
_NSORT = 256
_VIEWS = 4
_ROWS_PER_STEP = 1024
_PAD_F = 2.0**18


_G = 32
_L = 128


def _cmphalf(a, b, asc):
    if asc:
        return jnp.minimum(a, b), jnp.maximum(a, b)
    return jnp.maximum(a, b), jnp.minimum(a, b)


def _halfclean(t, k, j, asc, g):
    b_dim = t.shape[0]
    z = t.reshape(-1, 2 * j * g, _L)
    p, q = _cmphalf(z[:, : j * g], z[:, j * g :], asc)
    return jnp.concatenate([p, q], axis=1).reshape(b_dim, k * g, _L)


def _halfclean2(t, k, j, asc, g):
    b_dim = t.shape[0]
    z = t.reshape(-1, 2 * j * g, _L)
    p, q = _cmphalf(z[:, : j * g], z[:, j * g :], asc)
    h = (j // 2) * g
    outs = []
    for blk in (p, q):
        r, s = _cmphalf(blk[:, :h], blk[:, h:], asc)
        outs += [r, s]
    return jnp.concatenate(outs, axis=1).reshape(b_dim, k * g, _L)


def _bitonic_sort_grouped(x, n, g):
    k = 2
    while k <= n:
        js = []
        j = k // 2
        while j >= 1:
            js.append(j)
            j //= 2
        ops = []
        i = 0
        while i < len(js):
            if i + 1 < len(js):
                ops.append((_halfclean2, js[i]))
                i += 2
            else:
                ops.append((_halfclean, js[i]))
                i += 1
        nb = n // k
        for fn, j in ops:
            if nb == 1:
                x = fn(x.reshape(1, n * g, _L), k, j, True, g)
                x = x.reshape(n * g, _L)
            else:
                y = x.reshape(nb // 2, 2 * k * g, _L)
                asc = fn(y[:, : k * g], k, j, True, g)
                desc = fn(y[:, k * g :], k, j, False, g)
                x = jnp.concatenate([asc, desc], axis=1).reshape(n * g, _L)
        k *= 2
    return x


def _body(x_ref, o_ref):
    nat = x_ref[...].reshape(8, _L, _K)
    x = jnp.transpose(nat, (2, 0, 1))
    x = x.astype(jnp.float32)
    pad = jnp.full((_NSORT - _K, 8, _L), _PAD_F, jnp.float32)
    x = jnp.concatenate([x, pad], axis=0)
    s = _bitonic_sort_grouped(x.reshape(_NSORT * 8, _L), _NSORT, 8)
    s = s.reshape(_NSORT, 8, _L)
    neq = (s[1:_K] != s[: _K - 1]).astype(jnp.float32)
    uniq = 1.0 + jnp.sum(neq, axis=0)
    contrib = -0.25 * uniq

    @pl.when(pl.program_id(1) == 0)
    def _init():
        o_ref[0] = contrib

    @pl.when(pl.program_id(1) > 0)
    def _accum():
        o_ref[0] += contrib


def kernel(neighbors, images):
    del images
    n_rows, k = neighbors.shape
    batch = n_rows // _VIEWS
    n_steps = batch // _ROWS_PER_STEP

    out = pl.pallas_call(
        _body,
        grid=(n_steps, _VIEWS),
        in_specs=[
            pl.BlockSpec(
                (_ROWS_PER_STEP, k), lambda b, v: (v * n_steps + b, 0)
            )
        ],
        out_specs=pl.BlockSpec((1, 8, 128), lambda b, v: (b, 0, 0)),
        out_shape=jax.ShapeDtypeStruct((n_steps, 8, 128), jnp.float32),
        compiler_params=pltpu.CompilerParams(
            dimension_semantics=("parallel", "arbitrary"),
        ),
    )(neighbors)
    return out.reshape(batch)
